# baseline (speedup 1.0000x reference)
import functools

import functools

import jax
import jax.numpy as jnp
import numpy as np
from jax import lax
from jax.experimental import pallas as pl
from jax.experimental.pallas import tpu as pltpu
from jax.experimental.pallas import tpu_sc as plsc

STRIDE = 16
SIZES = (32.0, 64.0, 128.0, 256.0, 512.0)
PRE_NMS_TOP_N = 2000
POST_NMS_TOP_N = 1000
NMS_THRESH = 0.7
BBOX_XFORM_CLIP = float(np.log(1000.0 / 16.0))
KPAD = 2048


def _decode_nms_kernel(dx_ref, dy_ref, ew_ref, eh_ref, aw_ref, ah_ref,
                       acx_ref, acy_ref, sc_ref,
                       x1_ref, y1_ref, x2_ref, y2_ref, msk_ref):
    w = aw_ref[:]
    h = ah_ref[:]
    cx = acx_ref[:]
    cy = acy_ref[:]
    pcx = dx_ref[:] * w + cx
    pcy = dy_ref[:] * h + cy
    pw = ew_ref[:] * w
    ph = eh_ref[:] * h
    x1 = jnp.clip(pcx - 0.5 * pw, 0.0, 1023.0)
    y1 = jnp.clip(pcy - 0.5 * ph, 0.0, 1023.0)
    x2 = jnp.clip(pcx + 0.5 * pw - 1.0, 0.0, 1023.0)
    y2 = jnp.clip(pcy + 0.5 * ph - 1.0, 0.0, 1023.0)
    x1_ref[:] = x1
    y1_ref[:] = y1
    x2_ref[:] = x2
    y2_ref[:] = y2
    area = (x2 - x1 + 1.0) * (y2 - y1 + 1.0)

    x1r = jnp.transpose(x1)
    y1r = jnp.transpose(y1)
    x2r = jnp.transpose(x2)
    y2r = jnp.transpose(y2)
    arear = jnp.transpose(area)

    lt_x = jnp.maximum(x1, x1r)
    lt_y = jnp.maximum(y1, y1r)
    rb_x = jnp.minimum(x2, x2r)
    rb_y = jnp.minimum(y2, y2r)
    w = jnp.maximum(rb_x - lt_x + 1.0, 0.0)
    h = jnp.maximum(rb_y - lt_y + 1.0, 0.0)
    inter = w * h
    iou = inter / (area + arear - inter)

    jj = lax.broadcasted_iota(jnp.int32, (KPAD, KPAD), 0)
    ii = lax.broadcasted_iota(jnp.int32, (KPAD, KPAD), 1)
    valid = (jj < ii) & (ii < PRE_NMS_TOP_N) & (jj < PRE_NMS_TOP_N)
    m = jnp.where((iou > NMS_THRESH) & valid, 1.0, 0.0)

    keep0 = jnp.ones((8, KPAD), dtype=jnp.float32)

    def body(carry):
        keep, _ = carry
        s = jnp.dot(keep, m, preferred_element_type=jnp.float32)
        new = jnp.where(s == 0.0, 1.0, 0.0)
        changed = jnp.sum(jnp.abs(new - keep)) > 0.0
        return new, changed

    def cond(carry):
        return carry[1]

    keep, _ = lax.while_loop(cond, body, (keep0, jnp.bool_(True)))
    keep_row = keep[0:1, :] > 0.5
    pos = lax.broadcasted_iota(jnp.int32, (1, KPAD), 1)
    sc_row = jnp.transpose(sc_ref[:])
    msk_ref[:] = jnp.where(pos < PRE_NMS_TOP_N,
                           jnp.where(keep_row, sc_row, -1.0), -2.0)


def _decode_nms_pallas(dx, dy, ew, eh, aw, ah, acx, acy, scores):
    col = lambda v: v.reshape(KPAD, 1)
    outs = pl.pallas_call(
        _decode_nms_kernel,
        out_shape=(jax.ShapeDtypeStruct((KPAD, 1), jnp.float32),) * 4
        + (jax.ShapeDtypeStruct((1, KPAD), jnp.float32),),
    )(col(dx), col(dy), col(ew), col(eh), col(aw), col(ah), col(acx),
      col(acy), col(scores))
    return outs


def _final_sort_kernel(m_ref, x1_ref, y1_ref, x2_ref, y2_ref, sc_ref,
                       ox1_ref, oy1_ref, ox2_ref, oy2_ref, osc_ref):
    s = m_ref[:]
    pos = (lax.broadcasted_iota(jnp.int32, (16, 128), 0) * 128
           + lax.broadcasted_iota(jnp.int32, (16, 128), 1))
    ps = [pos, x1_ref[:], y1_ref[:], x2_ref[:], y2_ref[:], sc_ref[:]]
    k = 2
    while k <= 2048:
        j = k // 2
        while j >= 1:
            s, ps = _cmpex(s, ps, j, lambda i, kk=k: (i & kk) == 0)
            j //= 2
        k *= 2
    ox1_ref[:] = ps[1]
    oy1_ref[:] = ps[2]
    ox2_ref[:] = ps[3]
    oy2_ref[:] = ps[4]
    osc_ref[:] = ps[5]


def _final_sort_pallas(masked, x1, y1, x2, y2, scores):
    grid16 = lambda v: v.reshape(16, 128)
    return pl.pallas_call(
        _final_sort_kernel,
        out_shape=(jax.ShapeDtypeStruct((16, 128), jnp.float32),) * 5,
    )(grid16(masked), grid16(x1), grid16(y1), grid16(x2), grid16(y2),
      grid16(scores))


def _sc_gather8(tables, idx):
    mesh = plsc.VectorSubcoreMesh(core_axis_name="c", subcore_axis_name="s")

    @functools.partial(
        pl.kernel,
        out_type=tuple(jax.ShapeDtypeStruct((KPAD,), jnp.float32)
                       for _ in range(8)),
        mesh=mesh,
        scratch_types=[pltpu.VMEM((64,), jnp.int32),
                       pltpu.VMEM((64,), jnp.float32),
                       pltpu.SemaphoreType.DMA],
    )
    def k(t0, t1, t2, t3, t4, t5, t6, t7, idx_hbm,
          o0, o1, o2, o3, o4, o5, o6, o7, idx_v, buf_v, sem):
        wid = lax.axis_index("s") * 2 + lax.axis_index("c")
        base = wid * 64
        pltpu.sync_copy(idx_hbm.at[pl.ds(base, 64)], idx_v)
        for tab, out in ((t0, o0), (t1, o1), (t2, o2), (t3, o3),
                         (t4, o4), (t5, o5), (t6, o6), (t7, o7)):
            pltpu.async_copy(tab.at[idx_v], buf_v, sem).wait()
            pltpu.sync_copy(buf_v, out.at[pl.ds(base, 64)])

    return k(*tables, idx)


def _conv_head_kernel(f_ref, w9_ref, cb_ref, hw_ref, hb_ref, out_ref):
    acc = jnp.zeros((4224, 256), dtype=jnp.float32)
    for t in range(9):
        off = (t // 3) * 66 + (t % 3)
        acc = acc + jnp.dot(f_ref[off:off + 4224, :], w9_ref[t],
                            preferred_element_type=jnp.float32)
    act = jax.nn.relu(acc + cb_ref[0][None, :])
    out_ref[:] = jnp.dot(act, hw_ref[:],
                         preferred_element_type=jnp.float32) + hb_ref[0][None, :]


def _conv_head_pallas(features, conv_w, conv_b, cls_w, cls_b, bbox_w, bbox_b):
    feat = jnp.transpose(features[0], (1, 2, 0))
    fpad = jnp.pad(feat, ((1, 1), (1, 1), (0, 0))).reshape(4356, 256)
    fpad = jnp.pad(fpad, ((0, 4), (0, 0)))
    w9 = jnp.transpose(conv_w, (2, 3, 1, 0)).reshape(9, 256, 256)
    hw = jnp.zeros((256, 128), jnp.float32)
    hw = hw.at[:, 0:5].set(jnp.transpose(cls_w[:, :, 0, 0]))
    hw = hw.at[:, 8:28].set(jnp.transpose(bbox_w[:, :, 0, 0]))
    hb = jnp.zeros((1, 128), jnp.float32)
    hb = hb.at[0, 0:5].set(cls_b)
    hb = hb.at[0, 8:28].set(bbox_b)
    return pl.pallas_call(
        _conv_head_kernel,
        out_shape=jax.ShapeDtypeStruct((4224, 128), jnp.float32),
    )(fpad, w9, conv_b.reshape(1, 256), hw, hb)


def _roll(x, sh, axis):
    n = x.shape[axis]
    return pltpu.roll(x, sh % n, axis)


def _cmpex(s, ps, j, asc_of):
    R = s.shape[0]
    row = lax.broadcasted_iota(jnp.int32, (R, 128), 0)
    lane = lax.broadcasted_iota(jnp.int32, (R, 128), 1)
    i_arr = row * 128 + lane
    if j >= 128:
        jr = j // 128
        is_A = (row & jr) == 0
        part = lambda x: jnp.where(is_A, _roll(x, -jr, 0), _roll(x, jr, 0))
    else:
        is_A = (lane & j) == 0
        part = lambda x: jnp.where(is_A, _roll(x, -j, 1), _roll(x, j, 1))
    s_ = part(s)
    ps_ = [part(x) for x in ps]
    less_xp = (s > s_) | ((s == s_) & (ps[0] < ps_[0]))
    asc = asc_of(i_arr)
    cond = ~(less_xp ^ ~(asc ^ is_A))
    return (jnp.where(cond, s, s_),
            [jnp.where(cond, x, x_) for x, x_ in zip(ps, ps_)])


def _winner_half(s, ps):
    R = s.shape[0]
    G = R // 32
    halves = lambda x: (x.reshape(G, 2, 16, 128)[:, 0],
                        x.reshape(G, 2, 16, 128)[:, 1])
    As, Bs = halves(s)
    Ap0, Bp0 = halves(ps[0])
    less_ab = (As > Bs) | ((As == Bs) & (Ap0 < Bp0))
    pick = lambda A, B: jnp.where(less_ab, A, B).reshape(R // 2, 128)
    return pick(As, Bs), [pick(*halves(x)) for x in ps]


def _topk_sort_kernel(s_ref, p_ref, os_ref, op_ref):
    s = s_ref[:]
    ps = [p_ref[:]]
    k = 2
    while k <= 2048:
        j = k // 2
        while j >= 1:
            s, ps = _cmpex(s, ps, j, lambda i, kk=k: (i & kk) == 0)
            j //= 2
        k *= 2
    while s.shape[0] > 16:
        s, ps = _winner_half(s, ps)
        j = 1024
        while j >= 1:
            s, ps = _cmpex(s, ps, j, lambda i: (i & 2048) == 0)
            j //= 2
    os_ref[:] = s
    op_ref[:] = ps[0]


def _topk_pallas(scores_flat, pack_flat):
    s = jnp.full((32768,), -1.0, jnp.float32).at[:21120].set(scores_flat)
    pq = jnp.concatenate([pack_flat, 40000 + jnp.arange(32768 - 21120,
                                                        dtype=jnp.int32)])
    return pl.pallas_call(
        _topk_sort_kernel,
        out_shape=(jax.ShapeDtypeStruct((16, 128), jnp.float32),
                   jax.ShapeDtypeStruct((16, 128), jnp.int32)),
    )(s.reshape(256, 128), pq.reshape(256, 128))


def _make_anchors(H, W):
    sizes = np.array(SIZES, dtype=np.float64)
    cell = np.stack([-(sizes - 1) / 2.0, -(sizes - 1) / 2.0,
                     (sizes - 1) / 2.0, (sizes - 1) / 2.0], axis=1)
    shift_x = np.arange(W, dtype=np.float64) * STRIDE
    shift_y = np.arange(H, dtype=np.float64) * STRIDE
    sy, sx = np.meshgrid(shift_y, shift_x, indexing="ij")
    shifts = np.stack([sx.ravel(), sy.ravel(), sx.ravel(), sy.ravel()], axis=1)
    anchors = (shifts[:, None, :] + cell[None, :, :]).reshape(-1, 4)
    return jnp.asarray(anchors, dtype=jnp.float32)


def kernel(images, features, conv_w, conv_b, cls_w, cls_b, bbox_w, bbox_b):
    out = _conv_head_pallas(features, conv_w, conv_b, cls_w, cls_b,
                            bbox_w, bbox_b)
    obj = out[:, 0:5].reshape(-1)
    regf = out[:, 8:28].reshape(4224, 5, 4).reshape(21120, 4)
    ar = jnp.arange(21120, dtype=jnp.int32)
    valid = (ar // 5) % 66 < 64
    scores = jnp.where(valid, jax.nn.sigmoid(obj), -1.0)
    pack = (ar // 5) * 8 + ar % 5
    s_sorted, p_sorted = _topk_pallas(scores, pack)
    top_scores = s_sorted.reshape(KPAD)
    f_idx = ((p_sorted >> 3) * 5 + (p_sorted & 7)).reshape(KPAD)
    anch = _make_anchors(64, 66)
    aw_t = anch[:, 2] - anch[:, 0] + 1.0
    ah_t = anch[:, 3] - anch[:, 1] + 1.0
    acx_t = anch[:, 0] + 0.5 * aw_t
    acy_t = anch[:, 1] + 0.5 * ah_t
    ew_t = jnp.exp(jnp.minimum(regf[:, 2], BBOX_XFORM_CLIP))
    eh_t = jnp.exp(jnp.minimum(regf[:, 3], BBOX_XFORM_CLIP))
    g = _sc_gather8([regf[:, 0], regf[:, 1], ew_t, eh_t,
                     aw_t, ah_t, acx_t, acy_t], f_idx)
    x1, y1, x2, y2, masked = _decode_nms_pallas(*g, top_scores)
    sx1, sy1, sx2, sy2, ssc = _final_sort_pallas(
        masked.reshape(KPAD), x1.reshape(KPAD), y1.reshape(KPAD),
        x2.reshape(KPAD), y2.reshape(KPAD), top_scores)
    n = POST_NMS_TOP_N
    return jnp.stack([sx1.reshape(KPAD)[:n], sy1.reshape(KPAD)[:n],
                      sx2.reshape(KPAD)[:n], sy2.reshape(KPAD)[:n],
                      ssc.reshape(KPAD)[:n]], axis=1)


# speedup vs baseline: 1.0561x; 1.0561x over previous
import functools

import jax
import jax.numpy as jnp
import numpy as np
from jax import lax
from jax.experimental import pallas as pl
from jax.experimental.pallas import tpu as pltpu
from jax.experimental.pallas import tpu_sc as plsc

STRIDE = 16
SIZES = (32.0, 64.0, 128.0, 256.0, 512.0)
PRE_NMS_TOP_N = 2000
POST_NMS_TOP_N = 1000
NMS_THRESH = 0.7
BBOX_XFORM_CLIP = float(np.log(1000.0 / 16.0))
KPAD = 2048


def _decode_nms_kernel(dx_ref, dy_ref, ew_ref, eh_ref, pk_ref, sc_ref,
                       x1_ref, y1_ref, x2_ref, y2_ref, msk_ref):
    pk = pk_ref[:]
    r = pk >> 3
    a = pk & 7
    rf = r.astype(jnp.float32)
    hh = jnp.floor((rf + 0.5) * (1.0 / 66.0))
    wf = rf - 66.0 * hh
    size = jnp.where(a == 0, 32.0,
                     jnp.where(a == 1, 64.0,
                               jnp.where(a == 2, 128.0,
                                         jnp.where(a == 3, 256.0, 512.0))))
    w = size
    h = size
    cx = wf * 16.0 + 0.5
    cy = hh * 16.0 + 0.5
    pcx = dx_ref[:] * w + cx
    pcy = dy_ref[:] * h + cy
    pw = ew_ref[:] * w
    ph = eh_ref[:] * h
    x1 = jnp.clip(pcx - 0.5 * pw, 0.0, 1023.0)
    y1 = jnp.clip(pcy - 0.5 * ph, 0.0, 1023.0)
    x2 = jnp.clip(pcx + 0.5 * pw - 1.0, 0.0, 1023.0)
    y2 = jnp.clip(pcy + 0.5 * ph - 1.0, 0.0, 1023.0)
    x1_ref[:] = x1
    y1_ref[:] = y1
    x2_ref[:] = x2
    y2_ref[:] = y2
    area = (x2 - x1 + 1.0) * (y2 - y1 + 1.0)

    x1r = jnp.transpose(x1)
    y1r = jnp.transpose(y1)
    x2r = jnp.transpose(x2)
    y2r = jnp.transpose(y2)
    arear = jnp.transpose(area)

    lt_x = jnp.maximum(x1, x1r)
    lt_y = jnp.maximum(y1, y1r)
    rb_x = jnp.minimum(x2, x2r)
    rb_y = jnp.minimum(y2, y2r)
    w = jnp.maximum(rb_x - lt_x + 1.0, 0.0)
    h = jnp.maximum(rb_y - lt_y + 1.0, 0.0)
    inter = w * h
    iou = inter / (area + arear - inter)

    jj = lax.broadcasted_iota(jnp.int32, (KPAD, KPAD), 0)
    ii = lax.broadcasted_iota(jnp.int32, (KPAD, KPAD), 1)
    valid = (jj < ii) & (ii < PRE_NMS_TOP_N) & (jj < PRE_NMS_TOP_N)
    m = jnp.where((iou > NMS_THRESH) & valid, 1.0, 0.0).astype(jnp.bfloat16)

    keep0 = jnp.ones((8, KPAD), dtype=jnp.bfloat16)

    def body(carry):
        keep, _ = carry
        s = jnp.dot(keep, m, preferred_element_type=jnp.float32)
        new = jnp.where(s == 0.0, 1.0, 0.0)
        changed = jnp.sum(jnp.abs(new - keep.astype(jnp.float32))) > 0.0
        return new.astype(jnp.bfloat16), changed

    def cond(carry):
        return carry[1]

    keep, _ = lax.while_loop(cond, body, (keep0, jnp.bool_(True)))
    keep_row = keep[0:1, :] > 0.5
    pos = lax.broadcasted_iota(jnp.int32, (1, KPAD), 1)
    sc_row = jnp.transpose(sc_ref[:])
    msk_ref[:] = jnp.where(pos < PRE_NMS_TOP_N,
                           jnp.where(keep_row, sc_row, -1.0), -2.0)


def _decode_nms_pallas(dx, dy, ew, eh, pack, scores):
    col = lambda v: v.reshape(KPAD, 1)
    outs = pl.pallas_call(
        _decode_nms_kernel,
        out_shape=(jax.ShapeDtypeStruct((KPAD, 1), jnp.float32),) * 4
        + (jax.ShapeDtypeStruct((1, KPAD), jnp.float32),),
    )(col(dx), col(dy), col(ew), col(eh), col(pack), col(scores))
    return outs


def _final_sort_kernel(m_ref, x1_ref, y1_ref, x2_ref, y2_ref, sc_ref,
                       ox1_ref, oy1_ref, ox2_ref, oy2_ref, osc_ref):
    s = m_ref[:]
    pos = (lax.broadcasted_iota(jnp.int32, (16, 128), 0) * 128
           + lax.broadcasted_iota(jnp.int32, (16, 128), 1))
    ps = [pos, x1_ref[:], y1_ref[:], x2_ref[:], y2_ref[:], sc_ref[:]]
    k = 2
    while k <= 2048:
        j = k // 2
        while j >= 1:
            s, ps = _cmpex(s, ps, j, lambda i, kk=k: (i & kk) == 0)
            j //= 2
        k *= 2
    ox1_ref[:] = ps[1]
    oy1_ref[:] = ps[2]
    ox2_ref[:] = ps[3]
    oy2_ref[:] = ps[4]
    osc_ref[:] = ps[5]


def _final_sort_pallas(masked, x1, y1, x2, y2, scores):
    grid16 = lambda v: v.reshape(16, 128)
    return pl.pallas_call(
        _final_sort_kernel,
        out_shape=(jax.ShapeDtypeStruct((16, 128), jnp.float32),) * 5,
    )(grid16(masked), grid16(x1), grid16(y1), grid16(x2), grid16(y2),
      grid16(scores))


def _sc_gather4(tables, idx):
    mesh = plsc.VectorSubcoreMesh(core_axis_name="c", subcore_axis_name="s")

    @functools.partial(
        pl.kernel,
        out_type=tuple(jax.ShapeDtypeStruct((KPAD,), jnp.float32)
                       for _ in range(4)),
        mesh=mesh,
        scratch_types=[pltpu.VMEM((64,), jnp.int32)]
        + [pltpu.VMEM((64,), jnp.float32)] * 4
        + [pltpu.SemaphoreType.DMA],
    )
    def k(t0, t1, t2, t3, idx_hbm, o0, o1, o2, o3,
          idx_v, b0, b1, b2, b3, sem):
        wid = lax.axis_index("s") * 2 + lax.axis_index("c")
        base = wid * 64
        pltpu.sync_copy(idx_hbm.at[pl.ds(base, 64)], idx_v)
        bufs = (b0, b1, b2, b3)
        copies = [pltpu.async_copy(tab.at[idx_v], buf, sem)
                  for tab, buf in zip((t0, t1, t2, t3), bufs)]
        for c in copies:
            c.wait()
        for buf, out in zip(bufs, (o0, o1, o2, o3)):
            pltpu.sync_copy(buf, out.at[pl.ds(base, 64)])

    return k(*tables, idx)


def _conv_head_kernel(f_ref, w9_ref, cb_ref, hw_ref, hb_ref, out_ref):
    acc = jnp.zeros((4224, 256), dtype=jnp.float32)
    for t in range(9):
        off = (t // 3) * 66 + (t % 3)
        acc = acc + jnp.dot(f_ref[off:off + 4224, :], w9_ref[t],
                            preferred_element_type=jnp.float32)
    act = jax.nn.relu(acc + cb_ref[0][None, :])
    out_ref[:] = jnp.dot(act, hw_ref[:],
                         preferred_element_type=jnp.float32) + hb_ref[0][None, :]


def _conv_head_pallas(features, conv_w, conv_b, cls_w, cls_b, bbox_w, bbox_b):
    feat = jnp.transpose(features[0], (1, 2, 0))
    fpad = jnp.pad(feat, ((1, 1), (1, 1), (0, 0))).reshape(4356, 256)
    fpad = jnp.pad(fpad, ((0, 4), (0, 0)))
    w9 = jnp.transpose(conv_w, (2, 3, 1, 0)).reshape(9, 256, 256)
    bw2 = bbox_w[:, :, 0, 0]
    hw = jnp.zeros((256, 128), jnp.float32)
    hw = hw.at[:, 0:5].set(jnp.transpose(cls_w[:, :, 0, 0]))
    hb = jnp.zeros((1, 128), jnp.float32)
    hb = hb.at[0, 0:5].set(cls_b)
    for c in range(4):
        hw = hw.at[:, 8 + 8 * c:13 + 8 * c].set(jnp.transpose(bw2[c::4]))
        hb = hb.at[0, 8 + 8 * c:13 + 8 * c].set(bbox_b[c::4])
    return pl.pallas_call(
        _conv_head_kernel,
        out_shape=jax.ShapeDtypeStruct((4224, 128), jnp.float32),
    )(fpad, w9, conv_b.reshape(1, 256), hw, hb)


def _roll(x, sh, axis):
    n = x.shape[axis]
    return pltpu.roll(x, sh % n, axis)


def _cmpex(s, ps, j, asc_of):
    R = s.shape[0]
    row = lax.broadcasted_iota(jnp.int32, (R, 128), 0)
    lane = lax.broadcasted_iota(jnp.int32, (R, 128), 1)
    i_arr = row * 128 + lane
    if j >= 128:
        jr = j // 128
        is_A = (row & jr) == 0
        part = lambda x: jnp.where(is_A, _roll(x, -jr, 0), _roll(x, jr, 0))
    else:
        is_A = (lane & j) == 0
        part = lambda x: jnp.where(is_A, _roll(x, -j, 1), _roll(x, j, 1))
    s_ = part(s)
    ps_ = [part(x) for x in ps]
    less_xp = (s > s_) | ((s == s_) & (ps[0] < ps_[0]))
    asc = asc_of(i_arr)
    cond = ~(less_xp ^ ~(asc ^ is_A))
    return (jnp.where(cond, s, s_),
            [jnp.where(cond, x, x_) for x, x_ in zip(ps, ps_)])


def _winner_half(s, ps):
    R = s.shape[0]
    G = R // 32
    halves = lambda x: (x.reshape(G, 2, 16, 128)[:, 0],
                        x.reshape(G, 2, 16, 128)[:, 1])
    As, Bs = halves(s)
    Ap0, Bp0 = halves(ps[0])
    less_ab = (As > Bs) | ((As == Bs) & (Ap0 < Bp0))
    pick = lambda A, B: jnp.where(less_ab, A, B).reshape(R // 2, 128)
    return pick(As, Bs), [pick(*halves(x)) for x in ps]


def _topk_sort_kernel(s_ref, p_ref, os_ref, op_ref):
    s = s_ref[:]
    ps = [p_ref[:]]
    k = 2
    while k <= 2048:
        j = k // 2
        while j >= 1:
            s, ps = _cmpex(s, ps, j, lambda i, kk=k: (i & kk) == 0)
            j //= 2
        k *= 2
    while s.shape[0] > 16:
        s, ps = _winner_half(s, ps)
        j = 1024
        while j >= 1:
            s, ps = _cmpex(s, ps, j, lambda i: (i & 2048) == 0)
            j //= 2
    os_ref[:] = s
    op_ref[:] = ps[0]


def _topk_pallas(scores_flat, pack_flat):
    s = jnp.full((32768,), -1.0, jnp.float32).at[:21120].set(scores_flat)
    pq = jnp.concatenate([pack_flat, 40000 + jnp.arange(32768 - 21120,
                                                        dtype=jnp.int32)])
    return pl.pallas_call(
        _topk_sort_kernel,
        out_shape=(jax.ShapeDtypeStruct((16, 128), jnp.float32),
                   jax.ShapeDtypeStruct((16, 128), jnp.int32)),
    )(s.reshape(256, 128), pq.reshape(256, 128))


def _make_anchors(H, W):
    sizes = np.array(SIZES, dtype=np.float64)
    cell = np.stack([-(sizes - 1) / 2.0, -(sizes - 1) / 2.0,
                     (sizes - 1) / 2.0, (sizes - 1) / 2.0], axis=1)
    shift_x = np.arange(W, dtype=np.float64) * STRIDE
    shift_y = np.arange(H, dtype=np.float64) * STRIDE
    sy, sx = np.meshgrid(shift_y, shift_x, indexing="ij")
    shifts = np.stack([sx.ravel(), sy.ravel(), sx.ravel(), sy.ravel()], axis=1)
    anchors = (shifts[:, None, :] + cell[None, :, :]).reshape(-1, 4)
    return jnp.asarray(anchors, dtype=jnp.float32)


def kernel(images, features, conv_w, conv_b, cls_w, cls_b, bbox_w, bbox_b):
    out = _conv_head_pallas(features, conv_w, conv_b, cls_w, cls_b,
                            bbox_w, bbox_b)
    obj = out[:, 0:5].reshape(-1)
    ar = jnp.arange(21120, dtype=jnp.int32)
    valid = (ar // 5) % 66 < 64
    scores = jnp.where(valid, jax.nn.sigmoid(obj), -1.0)
    pack = (ar // 5) * 8 + ar % 5
    s_sorted, p_sorted = _topk_pallas(scores, pack)
    top_scores = s_sorted.reshape(KPAD)
    p_sorted = p_sorted.reshape(KPAD)
    f_idx = (p_sorted >> 3) * 5 + (p_sorted & 7)
    dx_t = out[:, 8:13].reshape(21120)
    dy_t = out[:, 16:21].reshape(21120)
    ew_t = jnp.exp(jnp.minimum(out[:, 24:29].reshape(21120), BBOX_XFORM_CLIP))
    eh_t = jnp.exp(jnp.minimum(out[:, 32:37].reshape(21120), BBOX_XFORM_CLIP))
    g = _sc_gather4([dx_t, dy_t, ew_t, eh_t], f_idx)
    x1, y1, x2, y2, masked = _decode_nms_pallas(*g, p_sorted, top_scores)
    sx1, sy1, sx2, sy2, ssc = _final_sort_pallas(
        masked.reshape(KPAD), x1.reshape(KPAD), y1.reshape(KPAD),
        x2.reshape(KPAD), y2.reshape(KPAD), top_scores)
    n = POST_NMS_TOP_N
    return jnp.stack([sx1.reshape(KPAD)[:n], sy1.reshape(KPAD)[:n],
                      sx2.reshape(KPAD)[:n], sy2.reshape(KPAD)[:n],
                      ssc.reshape(KPAD)[:n]], axis=1)


# speedup vs baseline: 1.2331x; 1.1676x over previous
import functools

import jax
import jax.numpy as jnp
import numpy as np
from jax import lax
from jax.experimental import pallas as pl
from jax.experimental.pallas import tpu as pltpu
from jax.experimental.pallas import tpu_sc as plsc

STRIDE = 16
SIZES = (32.0, 64.0, 128.0, 256.0, 512.0)
PRE_NMS_TOP_N = 2000
POST_NMS_TOP_N = 1000
NMS_THRESH = 0.7
BBOX_XFORM_CLIP = float(np.log(1000.0 / 16.0))
KPAD = 2048


def _decode_nms_kernel(dx_ref, dy_ref, ew_ref, eh_ref, pk_ref, sc_ref,
                       x1_ref, y1_ref, x2_ref, y2_ref, msk_ref):
    pk = pk_ref[:]
    r = pk >> 3
    a = pk & 7
    rf = r.astype(jnp.float32)
    hh = jnp.floor((rf + 0.5) * (1.0 / 66.0))
    wf = rf - 66.0 * hh
    size = jnp.where(a == 0, 32.0,
                     jnp.where(a == 1, 64.0,
                               jnp.where(a == 2, 128.0,
                                         jnp.where(a == 3, 256.0, 512.0))))
    w = size
    h = size
    cx = wf * 16.0 + 0.5
    cy = hh * 16.0 + 0.5
    pcx = dx_ref[:] * w + cx
    pcy = dy_ref[:] * h + cy
    pw = ew_ref[:] * w
    ph = eh_ref[:] * h
    x1 = jnp.clip(pcx - 0.5 * pw, 0.0, 1023.0)
    y1 = jnp.clip(pcy - 0.5 * ph, 0.0, 1023.0)
    x2 = jnp.clip(pcx + 0.5 * pw - 1.0, 0.0, 1023.0)
    y2 = jnp.clip(pcy + 0.5 * ph - 1.0, 0.0, 1023.0)
    x1_ref[:] = x1
    y1_ref[:] = y1
    x2_ref[:] = x2
    y2_ref[:] = y2
    area = (x2 - x1 + 1.0) * (y2 - y1 + 1.0)

    x1r = jnp.transpose(x1)
    y1r = jnp.transpose(y1)
    x2r = jnp.transpose(x2)
    y2r = jnp.transpose(y2)
    arear = jnp.transpose(area)

    lt_x = jnp.maximum(x1, x1r)
    lt_y = jnp.maximum(y1, y1r)
    rb_x = jnp.minimum(x2, x2r)
    rb_y = jnp.minimum(y2, y2r)
    w = jnp.maximum(rb_x - lt_x + 1.0, 0.0)
    h = jnp.maximum(rb_y - lt_y + 1.0, 0.0)
    inter = w * h
    iou = inter / (area + arear - inter)

    jj = lax.broadcasted_iota(jnp.int32, (KPAD, KPAD), 0)
    ii = lax.broadcasted_iota(jnp.int32, (KPAD, KPAD), 1)
    valid = (jj < ii) & (ii < PRE_NMS_TOP_N) & (jj < PRE_NMS_TOP_N)
    m = jnp.where((iou > NMS_THRESH) & valid, 1.0, 0.0).astype(jnp.bfloat16)

    keep0 = jnp.ones((8, KPAD), dtype=jnp.bfloat16)

    def body(carry):
        keep, _ = carry
        s = jnp.dot(keep, m, preferred_element_type=jnp.float32)
        new = jnp.where(s == 0.0, 1.0, 0.0)
        changed = jnp.sum(jnp.abs(new - keep.astype(jnp.float32))) > 0.0
        return new.astype(jnp.bfloat16), changed

    def cond(carry):
        return carry[1]

    keep, _ = lax.while_loop(cond, body, (keep0, jnp.bool_(True)))
    keep_row = keep[0:1, :] > 0.5
    pos = lax.broadcasted_iota(jnp.int32, (1, KPAD), 1)
    sc_row = jnp.transpose(sc_ref[:])
    msk_ref[:] = jnp.where(pos < PRE_NMS_TOP_N,
                           jnp.where(keep_row, sc_row, -1.0), -2.0)


def _decode_nms_pallas(dx, dy, ew, eh, pack, scores):
    col = lambda v: v.reshape(KPAD, 1)
    outs = pl.pallas_call(
        _decode_nms_kernel,
        out_shape=(jax.ShapeDtypeStruct((KPAD, 1), jnp.float32),) * 4
        + (jax.ShapeDtypeStruct((1, KPAD), jnp.float32),),
    )(col(dx), col(dy), col(ew), col(eh), col(pack), col(scores))
    return outs


def _final_sort_kernel(m_ref, x1_ref, y1_ref, x2_ref, y2_ref, sc_ref,
                       ox1_ref, oy1_ref, ox2_ref, oy2_ref, osc_ref):
    s = m_ref[:]
    pos = (lax.broadcasted_iota(jnp.int32, (16, 128), 0) * 128
           + lax.broadcasted_iota(jnp.int32, (16, 128), 1))
    ps = [pos, x1_ref[:], y1_ref[:], x2_ref[:], y2_ref[:], sc_ref[:]]
    k = 2
    while k <= 2048:
        j = k // 2
        while j >= 1:
            s, ps = _cmpex(s, ps, j, lambda i, kk=k: (i & kk) == 0)
            j //= 2
        k *= 2
    ox1_ref[:] = ps[1]
    oy1_ref[:] = ps[2]
    ox2_ref[:] = ps[3]
    oy2_ref[:] = ps[4]
    osc_ref[:] = ps[5]


def _final_sort_pallas(masked, x1, y1, x2, y2, scores):
    grid16 = lambda v: v.reshape(16, 128)
    return pl.pallas_call(
        _final_sort_kernel,
        out_shape=(jax.ShapeDtypeStruct((16, 128), jnp.float32),) * 5,
    )(grid16(masked), grid16(x1), grid16(y1), grid16(x2), grid16(y2),
      grid16(scores))


def _sc_gather_deltas(head, pack):
    mesh = plsc.VectorSubcoreMesh(core_axis_name="c", subcore_axis_name="s")

    @functools.partial(
        pl.kernel,
        out_type=tuple(jax.ShapeDtypeStruct((KPAD,), jnp.float32)
                       for _ in range(4)),
        mesh=mesh,
        scratch_types=[pltpu.VMEM((64,), jnp.int32)]
        + [pltpu.VMEM((64,), jnp.int32)] * 4
        + [pltpu.VMEM((64,), jnp.float32)] * 4
        + [pltpu.SemaphoreType.DMA],
    )
    def k(head_hbm, pack_hbm, o0, o1, o2, o3,
          pk_v, i0, i1, i2, i3, b0, b1, b2, b3, sem):
        wid = lax.axis_index("s") * 2 + lax.axis_index("c")
        base = wid * 64
        pltpu.sync_copy(pack_hbm.at[pl.ds(base, 64)], pk_v)
        idxs = (i0, i1, i2, i3)
        bufs = (b0, b1, b2, b3)
        for c2 in range(4):
            pk16 = pk_v[pl.ds(16 * c2, 16)]
            flat = (pk16 >> 3) * 128 + (pk16 & 7)
            for c in range(4):
                idxs[c][pl.ds(16 * c2, 16)] = flat + (8 + 8 * c)
        copies = [pltpu.async_copy(head_hbm.at[idx], buf, sem)
                  for idx, buf in zip(idxs, bufs)]
        for cp in copies:
            cp.wait()
        for buf, out in zip(bufs, (o0, o1, o2, o3)):
            pltpu.sync_copy(buf, out.at[pl.ds(base, 64)])

    return k(head.reshape(4224 * 128), pack)


def _conv_head_kernel(f_ref, w9_ref, cb_ref, hw_ref, hb_ref, out_ref):
    acc = jnp.zeros((4224, 256), dtype=jnp.float32)
    for t in range(9):
        off = (t // 3) * 66 + (t % 3)
        acc = acc + jnp.dot(f_ref[off:off + 4224, :], w9_ref[t],
                            preferred_element_type=jnp.float32)
    act = jax.nn.relu(acc + cb_ref[0][None, :])
    out_ref[:] = jnp.dot(act, hw_ref[:],
                         preferred_element_type=jnp.float32) + hb_ref[0][None, :]


def _conv_head_pallas(features, conv_w, conv_b, cls_w, cls_b, bbox_w, bbox_b):
    feat = jnp.transpose(features[0], (1, 2, 0))
    fpad = jnp.pad(feat, ((1, 1), (1, 1), (0, 0))).reshape(4356, 256)
    fpad = jnp.pad(fpad, ((0, 4), (0, 0)))
    w9 = jnp.transpose(conv_w, (2, 3, 1, 0)).reshape(9, 256, 256)
    bw2 = bbox_w[:, :, 0, 0]
    hw = jnp.zeros((256, 128), jnp.float32)
    hw = hw.at[:, 0:5].set(jnp.transpose(cls_w[:, :, 0, 0]))
    hb = jnp.zeros((1, 128), jnp.float32)
    hb = hb.at[0, 0:5].set(cls_b)
    for c in range(4):
        hw = hw.at[:, 8 + 8 * c:13 + 8 * c].set(jnp.transpose(bw2[c::4]))
        hb = hb.at[0, 8 + 8 * c:13 + 8 * c].set(bbox_b[c::4])
    return pl.pallas_call(
        _conv_head_kernel,
        out_shape=jax.ShapeDtypeStruct((4224, 128), jnp.float32),
    )(fpad, w9, conv_b.reshape(1, 256), hw, hb)


def _roll(x, sh, axis):
    n = x.shape[axis]
    return pltpu.roll(x, sh % n, axis)


def _cmpex(s, ps, j, asc_of):
    R = s.shape[0]
    row = lax.broadcasted_iota(jnp.int32, (R, 128), 0)
    lane = lax.broadcasted_iota(jnp.int32, (R, 128), 1)
    i_arr = row * 128 + lane
    if j >= 128:
        jr = j // 128
        is_A = (row & jr) == 0
        part = lambda x: jnp.where(is_A, _roll(x, -jr, 0), _roll(x, jr, 0))
    else:
        is_A = (lane & j) == 0
        part = lambda x: jnp.where(is_A, _roll(x, -j, 1), _roll(x, j, 1))
    s_ = part(s)
    ps_ = [part(x) for x in ps]
    less_xp = (s > s_) | ((s == s_) & (ps[0] < ps_[0]))
    asc = asc_of(i_arr)
    cond = ~(less_xp ^ ~(asc ^ is_A))
    return (jnp.where(cond, s, s_),
            [jnp.where(cond, x, x_) for x, x_ in zip(ps, ps_)])


def _winner_half(s, ps):
    R = s.shape[0]
    G = R // 32
    halves = lambda x: (x.reshape(G, 2, 16, 128)[:, 0],
                        x.reshape(G, 2, 16, 128)[:, 1])
    As, Bs = halves(s)
    Ap0, Bp0 = halves(ps[0])
    less_ab = (As > Bs) | ((As == Bs) & (Ap0 < Bp0))
    pick = lambda A, B: jnp.where(less_ab, A, B).reshape(R // 2, 128)
    return pick(As, Bs), [pick(*halves(x)) for x in ps]


def _topk_sort_kernel(s_ref, p_ref, os_ref, op_ref):
    s = s_ref[:]
    ps = [p_ref[:]]
    k = 2
    while k <= 2048:
        j = k // 2
        while j >= 1:
            s, ps = _cmpex(s, ps, j, lambda i, kk=k: (i & kk) == 0)
            j //= 2
        k *= 2
    while s.shape[0] > 16:
        s, ps = _winner_half(s, ps)
        j = 1024
        while j >= 1:
            s, ps = _cmpex(s, ps, j, lambda i: (i & 2048) == 0)
            j //= 2
    os_ref[:] = s
    op_ref[:] = ps[0]


def _topk_pallas(scores_flat, pack_flat):
    s = jnp.full((32768,), -1.0, jnp.float32).at[:21120].set(scores_flat)
    pq = jnp.concatenate([pack_flat, 40000 + jnp.arange(32768 - 21120,
                                                        dtype=jnp.int32)])
    return pl.pallas_call(
        _topk_sort_kernel,
        out_shape=(jax.ShapeDtypeStruct((16, 128), jnp.float32),
                   jax.ShapeDtypeStruct((16, 128), jnp.int32)),
    )(s.reshape(256, 128), pq.reshape(256, 128))


def _make_anchors(H, W):
    sizes = np.array(SIZES, dtype=np.float64)
    cell = np.stack([-(sizes - 1) / 2.0, -(sizes - 1) / 2.0,
                     (sizes - 1) / 2.0, (sizes - 1) / 2.0], axis=1)
    shift_x = np.arange(W, dtype=np.float64) * STRIDE
    shift_y = np.arange(H, dtype=np.float64) * STRIDE
    sy, sx = np.meshgrid(shift_y, shift_x, indexing="ij")
    shifts = np.stack([sx.ravel(), sy.ravel(), sx.ravel(), sy.ravel()], axis=1)
    anchors = (shifts[:, None, :] + cell[None, :, :]).reshape(-1, 4)
    return jnp.asarray(anchors, dtype=jnp.float32)


def kernel(images, features, conv_w, conv_b, cls_w, cls_b, bbox_w, bbox_b):
    out = _conv_head_pallas(features, conv_w, conv_b, cls_w, cls_b,
                            bbox_w, bbox_b)
    obj = out[:, 0:5].reshape(-1)
    ar = jnp.arange(21120, dtype=jnp.int32)
    valid = (ar // 5) % 66 < 64
    scores = jnp.where(valid, jax.nn.sigmoid(obj), -1.0)
    pack = (ar // 5) * 8 + ar % 5
    s_sorted, p_sorted = _topk_pallas(scores, pack)
    top_scores = s_sorted.reshape(KPAD)
    p_sorted = p_sorted.reshape(KPAD)
    dx, dy, dw, dh = _sc_gather_deltas(out, p_sorted)
    ew = jnp.exp(jnp.minimum(dw, BBOX_XFORM_CLIP))
    eh = jnp.exp(jnp.minimum(dh, BBOX_XFORM_CLIP))
    x1, y1, x2, y2, masked = _decode_nms_pallas(dx, dy, ew, eh,
                                                p_sorted, top_scores)
    sx1, sy1, sx2, sy2, ssc = _final_sort_pallas(
        masked.reshape(KPAD), x1.reshape(KPAD), y1.reshape(KPAD),
        x2.reshape(KPAD), y2.reshape(KPAD), top_scores)
    n = POST_NMS_TOP_N
    return jnp.stack([sx1.reshape(KPAD)[:n], sy1.reshape(KPAD)[:n],
                      sx2.reshape(KPAD)[:n], sy2.reshape(KPAD)[:n],
                      ssc.reshape(KPAD)[:n]], axis=1)


# speedup vs baseline: 1.3419x; 1.0882x over previous
import functools

import jax
import jax.numpy as jnp
import numpy as np
from jax import lax
from jax.experimental import pallas as pl
from jax.experimental.pallas import tpu as pltpu
from jax.experimental.pallas import tpu_sc as plsc

STRIDE = 16
SIZES = (32.0, 64.0, 128.0, 256.0, 512.0)
PRE_NMS_TOP_N = 2000
POST_NMS_TOP_N = 1000
NMS_THRESH = 0.7
BBOX_XFORM_CLIP = float(np.log(1000.0 / 16.0))
KPAD = 2048


def _decode_nms_kernel(x1_ref, y1_ref, x2_ref, y2_ref, sc_ref, msk_ref):
    x1 = x1_ref[:]
    y1 = y1_ref[:]
    x2 = x2_ref[:]
    y2 = y2_ref[:]
    area = (x2 - x1 + 1.0) * (y2 - y1 + 1.0)

    x1r = jnp.transpose(x1)
    y1r = jnp.transpose(y1)
    x2r = jnp.transpose(x2)
    y2r = jnp.transpose(y2)
    arear = jnp.transpose(area)

    lt_x = jnp.maximum(x1, x1r)
    lt_y = jnp.maximum(y1, y1r)
    rb_x = jnp.minimum(x2, x2r)
    rb_y = jnp.minimum(y2, y2r)
    w = jnp.maximum(rb_x - lt_x + 1.0, 0.0)
    h = jnp.maximum(rb_y - lt_y + 1.0, 0.0)
    inter = w * h
    iou = inter / (area + arear - inter)

    jj = lax.broadcasted_iota(jnp.int32, (KPAD, KPAD), 0)
    ii = lax.broadcasted_iota(jnp.int32, (KPAD, KPAD), 1)
    valid = (jj < ii) & (ii < PRE_NMS_TOP_N) & (jj < PRE_NMS_TOP_N)
    m = jnp.where((iou > NMS_THRESH) & valid, 1.0, 0.0).astype(jnp.bfloat16)

    keep0 = jnp.ones((8, KPAD), dtype=jnp.bfloat16)

    def body(carry):
        keep, _ = carry
        s = jnp.dot(keep, m, preferred_element_type=jnp.float32)
        new = jnp.where(s == 0.0, 1.0, 0.0)
        changed = jnp.sum(jnp.abs(new - keep.astype(jnp.float32))) > 0.0
        return new.astype(jnp.bfloat16), changed

    def cond(carry):
        return carry[1]

    keep, _ = lax.while_loop(cond, body, (keep0, jnp.bool_(True)))
    keep_row = keep[0:1, :] > 0.5
    pos = lax.broadcasted_iota(jnp.int32, (1, KPAD), 1)
    sc_row = jnp.transpose(sc_ref[:])
    msk_ref[:] = jnp.where(pos < PRE_NMS_TOP_N,
                           jnp.where(keep_row, sc_row, -1.0), -2.0)


def _decode_nms_pallas(x1, y1, x2, y2, scores):
    col = lambda v: v.reshape(KPAD, 1)
    return pl.pallas_call(
        _decode_nms_kernel,
        out_shape=jax.ShapeDtypeStruct((1, KPAD), jnp.float32),
    )(col(x1), col(y1), col(x2), col(y2), col(scores))


def _final_sort_kernel(m_ref, x1_ref, y1_ref, x2_ref, y2_ref, sc_ref,
                       ox1_ref, oy1_ref, ox2_ref, oy2_ref, osc_ref):
    s = m_ref[:]
    pos = (lax.broadcasted_iota(jnp.int32, (16, 128), 0) * 128
           + lax.broadcasted_iota(jnp.int32, (16, 128), 1))
    ps = [pos, x1_ref[:], y1_ref[:], x2_ref[:], y2_ref[:], sc_ref[:]]
    k = 2
    while k <= 2048:
        j = k // 2
        while j >= 1:
            s, ps = _cmpex(s, ps, j, lambda i, kk=k: (i & kk) == 0)
            j //= 2
        k *= 2
    ox1_ref[:] = ps[1]
    oy1_ref[:] = ps[2]
    ox2_ref[:] = ps[3]
    oy2_ref[:] = ps[4]
    osc_ref[:] = ps[5]


def _final_sort_pallas(masked, x1, y1, x2, y2, scores):
    grid16 = lambda v: v.reshape(16, 128)
    return pl.pallas_call(
        _final_sort_kernel,
        out_shape=(jax.ShapeDtypeStruct((16, 128), jnp.float32),) * 5,
    )(grid16(masked), grid16(x1), grid16(y1), grid16(x2), grid16(y2),
      grid16(scores))


def _sc_gather_deltas(head, pack):
    mesh = plsc.VectorSubcoreMesh(core_axis_name="c", subcore_axis_name="s")

    @functools.partial(
        pl.kernel,
        out_type=tuple(jax.ShapeDtypeStruct((KPAD,), jnp.float32)
                       for _ in range(4)),
        mesh=mesh,
        scratch_types=[pltpu.VMEM((64,), jnp.int32)]
        + [pltpu.VMEM((64,), jnp.int32)] * 4
        + [pltpu.VMEM((64,), jnp.float32)] * 4
        + [pltpu.SemaphoreType.DMA],
    )
    def k(head_hbm, pack_hbm, o0, o1, o2, o3,
          pk_v, i0, i1, i2, i3, b0, b1, b2, b3, sem):
        wid = lax.axis_index("s") * 2 + lax.axis_index("c")
        base = wid * 64
        pltpu.sync_copy(pack_hbm.at[pl.ds(base, 64)], pk_v)
        idxs = (i0, i1, i2, i3)
        bufs = (b0, b1, b2, b3)
        for c2 in range(4):
            pk16 = pk_v[pl.ds(16 * c2, 16)]
            flat = (pk16 >> 3) * 128 + (pk16 & 7)
            for c in range(4):
                idxs[c][pl.ds(16 * c2, 16)] = flat + (8 + 8 * c)
        copies = [pltpu.async_copy(head_hbm.at[idx], buf, sem)
                  for idx, buf in zip(idxs, bufs)]
        for cp in copies:
            cp.wait()
        for buf, out in zip(bufs, (o0, o1, o2, o3)):
            pltpu.sync_copy(buf, out.at[pl.ds(base, 64)])

    return k(head.reshape(4224 * 128), pack)


def _conv_head_kernel(f_ref, w9_ref, cb_ref, hw_ref, hb_ref, out_ref):
    acc = jnp.zeros((4224, 256), dtype=jnp.float32)
    for t in range(9):
        off = (t // 3) * 66 + (t % 3)
        acc = acc + jnp.dot(f_ref[off:off + 4224, :], w9_ref[t],
                            preferred_element_type=jnp.float32)
    act = jax.nn.relu(acc + cb_ref[0][None, :])
    out_ref[:] = jnp.dot(act, hw_ref[:],
                         preferred_element_type=jnp.float32) + hb_ref[0][None, :]


def _conv_head_pallas(features, conv_w, conv_b, cls_w, cls_b, bbox_w, bbox_b):
    feat = jnp.transpose(features[0], (1, 2, 0))
    fpad = jnp.pad(feat, ((1, 1), (1, 1), (0, 0))).reshape(4356, 256)
    fpad = jnp.pad(fpad, ((0, 4), (0, 0)))
    w9 = jnp.transpose(conv_w, (2, 3, 1, 0)).reshape(9, 256, 256)
    bw2 = bbox_w[:, :, 0, 0]
    hw = jnp.zeros((256, 128), jnp.float32)
    hw = hw.at[:, 0:5].set(jnp.transpose(cls_w[:, :, 0, 0]))
    hb = jnp.zeros((1, 128), jnp.float32)
    hb = hb.at[0, 0:5].set(cls_b)
    for c in range(4):
        hw = hw.at[:, 8 + 8 * c:13 + 8 * c].set(jnp.transpose(bw2[c::4]))
        hb = hb.at[0, 8 + 8 * c:13 + 8 * c].set(bbox_b[c::4])
    return pl.pallas_call(
        _conv_head_kernel,
        out_shape=jax.ShapeDtypeStruct((4224, 128), jnp.float32),
    )(fpad, w9, conv_b.reshape(1, 256), hw, hb)


def _roll(x, sh, axis):
    n = x.shape[axis]
    return pltpu.roll(x, sh % n, axis)


def _cmpex(s, ps, j, asc_of):
    R = s.shape[0]
    row = lax.broadcasted_iota(jnp.int32, (R, 128), 0)
    lane = lax.broadcasted_iota(jnp.int32, (R, 128), 1)
    i_arr = row * 128 + lane
    if j >= 128:
        jr = j // 128
        is_A = (row & jr) == 0
        part = lambda x: jnp.where(is_A, _roll(x, -jr, 0), _roll(x, jr, 0))
    else:
        is_A = (lane & j) == 0
        part = lambda x: jnp.where(is_A, _roll(x, -j, 1), _roll(x, j, 1))
    s_ = part(s)
    ps_ = [part(x) for x in ps]
    less_xp = (s > s_) | ((s == s_) & (ps[0] < ps_[0]))
    asc = asc_of(i_arr)
    cond = ~(less_xp ^ ~(asc ^ is_A))
    return (jnp.where(cond, s, s_),
            [jnp.where(cond, x, x_) for x, x_ in zip(ps, ps_)])


def _winner_half(s, ps):
    R = s.shape[0]
    G = R // 32
    halves = lambda x: (x.reshape(G, 2, 16, 128)[:, 0],
                        x.reshape(G, 2, 16, 128)[:, 1])
    As, Bs = halves(s)
    Ap0, Bp0 = halves(ps[0])
    less_ab = (As > Bs) | ((As == Bs) & (Ap0 < Bp0))
    pick = lambda A, B: jnp.where(less_ab, A, B).reshape(R // 2, 128)
    return pick(As, Bs), [pick(*halves(x)) for x in ps]


def _topk_sort_kernel(s_ref, p_ref, os_ref, op_ref):
    s = s_ref[:]
    ps = [p_ref[:]]
    k = 2
    while k <= 2048:
        j = k // 2
        while j >= 1:
            s, ps = _cmpex(s, ps, j, lambda i, kk=k: (i & kk) == 0)
            j //= 2
        k *= 2
    while s.shape[0] > 16:
        s, ps = _winner_half(s, ps)
        j = 1024
        while j >= 1:
            s, ps = _cmpex(s, ps, j, lambda i: (i & 2048) == 0)
            j //= 2
    os_ref[:] = s
    op_ref[:] = ps[0]


def _topk_pallas(scores_flat, pack_flat):
    s = jnp.full((32768,), -1.0, jnp.float32).at[:21120].set(scores_flat)
    pq = jnp.concatenate([pack_flat, 40000 + jnp.arange(32768 - 21120,
                                                        dtype=jnp.int32)])
    return pl.pallas_call(
        _topk_sort_kernel,
        out_shape=(jax.ShapeDtypeStruct((16, 128), jnp.float32),
                   jax.ShapeDtypeStruct((16, 128), jnp.int32)),
    )(s.reshape(256, 128), pq.reshape(256, 128))


def _make_anchors(H, W):
    sizes = np.array(SIZES, dtype=np.float64)
    cell = np.stack([-(sizes - 1) / 2.0, -(sizes - 1) / 2.0,
                     (sizes - 1) / 2.0, (sizes - 1) / 2.0], axis=1)
    shift_x = np.arange(W, dtype=np.float64) * STRIDE
    shift_y = np.arange(H, dtype=np.float64) * STRIDE
    sy, sx = np.meshgrid(shift_y, shift_x, indexing="ij")
    shifts = np.stack([sx.ravel(), sy.ravel(), sx.ravel(), sy.ravel()], axis=1)
    anchors = (shifts[:, None, :] + cell[None, :, :]).reshape(-1, 4)
    return jnp.asarray(anchors, dtype=jnp.float32)


def kernel(images, features, conv_w, conv_b, cls_w, cls_b, bbox_w, bbox_b):
    out = _conv_head_pallas(features, conv_w, conv_b, cls_w, cls_b,
                            bbox_w, bbox_b)
    obj = out[:, 0:5].reshape(-1)
    ar = jnp.arange(21120, dtype=jnp.int32)
    valid = (ar // 5) % 66 < 64
    scores = jnp.where(valid, jax.nn.sigmoid(obj), -1.0)
    pack = (ar // 5) * 8 + ar % 5
    s_sorted, p_sorted = _topk_pallas(scores, pack)
    top_scores = s_sorted.reshape(KPAD)
    p_sorted = p_sorted.reshape(KPAD)
    dx, dy, dw, dh = _sc_gather_deltas(out, p_sorted)
    r = p_sorted >> 3
    a = p_sorted & 7
    wf = (r % 66).astype(jnp.float32)
    hf = (r // 66).astype(jnp.float32)
    size = jnp.asarray(SIZES, jnp.float32)[a]
    cx = wf * 16.0 + 0.5
    cy = hf * 16.0 + 0.5
    pcx = dx * size + cx
    pcy = dy * size + cy
    pw = jnp.exp(jnp.minimum(dw, BBOX_XFORM_CLIP)) * size
    ph = jnp.exp(jnp.minimum(dh, BBOX_XFORM_CLIP)) * size
    x1 = jnp.clip(pcx - 0.5 * pw, 0.0, 1023.0)
    y1 = jnp.clip(pcy - 0.5 * ph, 0.0, 1023.0)
    x2 = jnp.clip(pcx + 0.5 * pw - 1.0, 0.0, 1023.0)
    y2 = jnp.clip(pcy + 0.5 * ph - 1.0, 0.0, 1023.0)
    masked = _decode_nms_pallas(x1, y1, x2, y2, top_scores)
    sx1, sy1, sx2, sy2, ssc = _final_sort_pallas(
        masked.reshape(KPAD), x1, y1, x2, y2, top_scores)
    n = POST_NMS_TOP_N
    return jnp.stack([sx1.reshape(KPAD)[:n], sy1.reshape(KPAD)[:n],
                      sx2.reshape(KPAD)[:n], sy2.reshape(KPAD)[:n],
                      ssc.reshape(KPAD)[:n]], axis=1)
